# Initial kernel scaffold; baseline (speedup 1.0000x reference)
import jax
import jax.numpy as jnp
from jax.experimental import pallas as pl


def kernel(feat0, feat1, feat2, feat3, feat4, conv_w, conv_b, cls_w, cls_b, reg_w, reg_b, res_img_shape, pad_img_shape):
    raise NotImplementedError("write your pallas kernel here")


# speedup vs baseline: 29.0916x; 29.0916x over previous
import functools
import math

import jax
import jax.numpy as jnp
from jax.experimental import pallas as pl
from jax.experimental.pallas import tpu as pltpu

_STRIDES = (4, 8, 16, 32, 64)
_NMS_PRE = 1000
_NMS_POST = 1000
_NMS_THR = 0.7
_BLK = 128
_MAX_RATIO = math.log(1000.0 / 16.0)


def _head_kernel(xpf_ref, w9_ref, cb_ref, wh_ref, bh_ref, sc_ref, rg_ref, *, W, P, bm, nb):
    base = pl.program_id(0) * bm if nb > 1 else 0
    r = base + jax.lax.broadcasted_iota(jnp.int32, (bm, 1), 0)
    wcol = r % W
    acc = jnp.zeros((bm, 256), jnp.float32)
    xall = xpf_ref[pl.ds(base, bm + 2 * P), :]
    for ky in range(3):
        for kx in range(3):
            s = (ky - 1) * W + (kx - 1)
            xs = xall[P + s:P + s + bm, :]
            if kx == 0:
                xs = jnp.where(wcol == 0, 0.0, xs)
            elif kx == 2:
                xs = jnp.where(wcol == W - 1, 0.0, xs)
            acc = acc + jnp.dot(xs, w9_ref[ky * 3 + kx],
                                preferred_element_type=jnp.float32)
    hidden = jnp.maximum(acc + cb_ref[0:1, :], 0.0)
    head = jnp.dot(hidden, wh_ref[...], preferred_element_type=jnp.float32)
    head = head + bh_ref[0:1, :]
    sc_ref[...] = head[:, 0:3]
    rg_ref[...] = head[:, 3:15]


def _run_head(x, H, W, w9, cb2, wh, bh2):
    HW = H * W
    nb = 8 if HW >= 15360 else (2 if HW >= 3840 else 1)
    bm = HW // nb
    P = W + 8
    xpf = jnp.pad(x, ((P, P), (0, 0)))
    sc, rg = pl.pallas_call(
        functools.partial(_head_kernel, W=W, P=P, bm=bm, nb=nb),
        grid=(nb,),
        in_specs=[
            pl.BlockSpec((HW + 2 * P, 256), lambda i: (0, 0)),
            pl.BlockSpec((9, 256, 256), lambda i: (0, 0, 0)),
            pl.BlockSpec((1, 256), lambda i: (0, 0)),
            pl.BlockSpec((256, 16), lambda i: (0, 0)),
            pl.BlockSpec((1, 16), lambda i: (0, 0)),
        ],
        out_specs=[
            pl.BlockSpec((bm, 3), lambda i: (i, 0)),
            pl.BlockSpec((bm, 12), lambda i: (i, 0)),
        ],
        out_shape=[
            jax.ShapeDtypeStruct((HW, 3), jnp.float32),
            jax.ShapeDtypeStruct((HW, 12), jnp.float32),
        ],
    )(xpf, w9, cb2, wh, bh2)
    return sc.reshape(-1), rg.reshape(HW * 3, 4)


def _nms_kernel(ancp_ref, anct_ref, dltp_ref, dltt_ref, res_ref,
                boxes_ref, keep_ref, *, n):
    rh = res_ref[0:1, 0:1]
    rw = res_ref[0:1, 1:2]

    def dec(x1, y1, x2, y2, dx, dy, dw, dh):
        aw = x2 - x1
        ah = y2 - y1
        ax = (x1 + x2) * 0.5
        ay = (y1 + y2) * 0.5
        dw = jnp.clip(dw, -_MAX_RATIO, _MAX_RATIO)
        dh = jnp.clip(dh, -_MAX_RATIO, _MAX_RATIO)
        px = ax + dx * aw
        py = ay + dy * ah
        pw = aw * jnp.exp(dw)
        ph = ah * jnp.exp(dh)
        bx1 = jnp.minimum(jnp.maximum(px - pw * 0.5, 0.0), rw)
        by1 = jnp.minimum(jnp.maximum(py - ph * 0.5, 0.0), rh)
        bx2 = jnp.minimum(jnp.maximum(px + pw * 0.5, 0.0), rw)
        by2 = jnp.minimum(jnp.maximum(py + ph * 0.5, 0.0), rh)
        return bx1, by1, bx2, by2

    anct = anct_ref[...]
    dltt = dltt_ref[...]
    X1r, Y1r, X2r, Y2r = dec(anct[0:1, :], anct[1:2, :], anct[2:3, :],
                             anct[3:4, :], dltt[0:1, :], dltt[1:2, :],
                             dltt[2:3, :], dltt[3:4, :])
    areaR = (X2r - X1r) * (Y2r - Y1r)

    ancp = ancp_ref[...]
    dltp = dltp_ref[...]
    X1c, Y1c, X2c, Y2c = dec(ancp[:, 0:1], ancp[:, 1:2], ancp[:, 2:3],
                             ancp[:, 3:4], dltp[:, 0:1], dltp[:, 1:2],
                             dltp[:, 2:3], dltp[:, 3:4])
    areaC = (X2c - X1c) * (Y2c - Y1c)

    colidx = jax.lax.broadcasted_iota(jnp.int32, (1, n), 1)
    jj = jax.lax.broadcasted_iota(jnp.int32, (_BLK, _BLK), 0)
    ii = jax.lax.broadcasted_iota(jnp.int32, (_BLK, _BLK), 1)
    lowtri = (jj < ii).astype(jnp.float32)

    supp = jnp.zeros((1, n), jnp.float32)
    keeps = []
    for b in range(n // _BLK):
        s = b * _BLK
        px1 = X1c[s:s + _BLK, :]
        py1 = Y1c[s:s + _BLK, :]
        px2 = X2c[s:s + _BLK, :]
        py2 = Y2c[s:s + _BLK, :]
        pa = areaC[s:s + _BLK, :]
        ix1 = jnp.maximum(px1, X1r)
        iy1 = jnp.maximum(py1, Y1r)
        ix2 = jnp.minimum(px2, X2r)
        iy2 = jnp.minimum(py2, Y2r)
        inter = jnp.maximum(ix2 - ix1, 0.0) * jnp.maximum(iy2 - iy1, 0.0)
        union = pa + areaR - inter
        iou = inter / jnp.maximum(union, 1e-6)
        big = (iou > _NMS_THR).astype(jnp.float32)
        M = big[:, s:s + _BLK] * lowtri
        init = jnp.where(supp[:, s:s + _BLK] > 0.0, 0.0, 1.0)

        def cond(c):
            return c[1]

        def body(c):
            k, _ = c
            sup = jnp.dot(k, M, preferred_element_type=jnp.float32)
            k2 = jnp.where(sup > 0.0, 0.0, init)
            return (k2, jnp.any(k2 != k))

        keep_b, _ = jax.lax.while_loop(cond, body, (init, True))
        keeps.append(keep_b)
        contrib = jnp.dot(keep_b, big, preferred_element_type=jnp.float32)
        supp = supp + jnp.where(colidx >= s + _BLK, contrib, 0.0)

    keep_ref[...] = jnp.concatenate(keeps, axis=1)
    boxes_ref[...] = jnp.concatenate([X1c, Y1c, X2c, Y2c], axis=1)


def kernel(feat0, feat1, feat2, feat3, feat4, conv_w, conv_b, cls_w, cls_b,
           reg_w, reg_b, res_img_shape, pad_img_shape):
    feats = [feat0, feat1, feat2, feat3, feat4]
    w9 = jnp.transpose(conv_w, (2, 3, 1, 0)).reshape(9, 256, 256)
    cb2 = conv_b.reshape(1, 256)
    wh = jnp.concatenate([
        jnp.transpose(cls_w[:, :, 0, 0]),
        jnp.transpose(reg_w[:, :, 0, 0]),
        jnp.zeros((256, 1), jnp.float32),
    ], axis=1)
    bh2 = jnp.concatenate([cls_b, reg_b,
                           jnp.zeros((1,), jnp.float32)]).reshape(1, 16)

    all_s, all_anc, all_dlt = [], [], []
    for li, f in enumerate(feats):
        H, W = f.shape[2], f.shape[3]
        HW = H * W
        x = f[0].reshape(256, HW).T
        lg, rg = _run_head(x, H, W, w9, cb2, wh, bh2)
        sc = jax.nn.sigmoid(lg)
        k = min(_NMS_PRE, HW * 3)
        ts, ti = jax.lax.top_k(sc, k)
        dl = rg[ti]
        stride = float(_STRIDES[li])
        ratios = jnp.array([0.5, 1.0, 2.0], dtype=jnp.float32)
        hr = jnp.sqrt(ratios)
        wr = 1.0 / hr
        ws = (stride * 8.0) * wr
        hs = (stride * 8.0) * hr
        bax = jnp.stack([-ws / 2.0, -hs / 2.0, ws / 2.0, hs / 2.0], axis=1)
        cell = ti // 3
        a = ti % 3
        wv = (cell % W).astype(jnp.float32) * stride
        hv = (cell // W).astype(jnp.float32) * stride
        shift = jnp.stack([wv, hv, wv, hv], axis=1)
        anc = shift + bax[a]
        all_s.append(ts)
        all_anc.append(anc)
        all_dlt.append(dl)

    ss = jnp.concatenate(all_s)
    anc = jnp.concatenate(all_anc, axis=0)
    dlt = jnp.concatenate(all_dlt, axis=0)
    n0 = ss.shape[0]
    order = jnp.argsort(-ss)
    ss_s = ss[order]
    anc_s = anc[order]
    dlt_s = dlt[order]

    npad = ((n0 + _BLK - 1) // _BLK) * _BLK
    ancp = jnp.pad(anc_s, ((0, npad - n0), (0, 0)))
    dltp = jnp.pad(dlt_s, ((0, npad - n0), (0, 0)))
    anct = ancp.T
    dltt = dltp.T
    res = res_img_shape.astype(jnp.float32).reshape(1, 2)

    boxes, keepr = pl.pallas_call(
        functools.partial(_nms_kernel, n=npad),
        out_shape=[
            jax.ShapeDtypeStruct((npad, 4), jnp.float32),
            jax.ShapeDtypeStruct((1, npad), jnp.float32),
        ],
    )(ancp, anct, dltp, dltt, res)

    keep = keepr[0, :n0] > 0.0
    masked = jnp.where(keep, ss_s, -1.0)
    post_s, post_i = jax.lax.top_k(masked, _NMS_POST)
    out = jnp.concatenate([boxes[:n0][post_i], post_s[:, None]], axis=1)
    return out[None]
